# initial kernel scaffold (unmeasured)
import jax
import jax.numpy as jnp
from jax import lax
from jax.experimental import pallas as pl
from jax.experimental.pallas import tpu as pltpu

N_DEV = 4
EPS = 1e-5


def kernel(x, gamma):
    m, n_local = x.shape
    n_global = n_local * N_DEV
    rows = m // 128
    g2 = gamma.reshape(1, n_local)

    def body(x_ref, g_ref, out_ref, comm_ref, send_sems, recv_sems):
        my = lax.axis_index("i")

        barrier = pltpu.get_barrier_semaphore()
        for k in range(1, N_DEV):
            peer = lax.rem(my + k, N_DEV)
            pl.semaphore_signal(
                barrier, inc=1,
                device_id=(peer,), device_id_type=pl.DeviceIdType.MESH,
            )
        pl.semaphore_wait(barrier, N_DEV - 1)

        x3 = x_ref[:, :].reshape(rows, 128, n_local)
        comm_ref[0, :, :] = jnp.sum(x3 * x3, axis=2)

        rdmas = []
        for k in range(1, N_DEV):
            peer = lax.rem(my + k, N_DEV)
            rdma = pltpu.make_async_remote_copy(
                src_ref=comm_ref.at[0],
                dst_ref=comm_ref.at[N_DEV - k],
                send_sem=send_sems.at[k - 1],
                recv_sem=recv_sems.at[N_DEV - k],
                device_id=(peer,),
                device_id_type=pl.DeviceIdType.MESH,
            )
            rdma.start()
            rdmas.append(rdma)
        for rdma in rdmas:
            rdma.wait()

        total = (
            comm_ref[0, :, :] + comm_ref[1, :, :]
            + comm_ref[2, :, :] + comm_ref[3, :, :]
        )
        inv = lax.rsqrt(total / n_global + EPS)
        out3 = x3 * inv.reshape(rows, 128, 1) * g_ref[:, :]
        out_ref[:, :] = out3.reshape(m, n_local)

    return pl.pallas_call(
        body,
        out_shape=jax.ShapeDtypeStruct((m, n_local), x.dtype),
        in_specs=[
            pl.BlockSpec(memory_space=pltpu.VMEM),
            pl.BlockSpec(memory_space=pltpu.VMEM),
        ],
        out_specs=pl.BlockSpec(memory_space=pltpu.VMEM),
        scratch_shapes=[
            pltpu.VMEM((N_DEV, rows, 128), jnp.float32),
            pltpu.SemaphoreType.DMA((N_DEV - 1,)),
            pltpu.SemaphoreType.DMA((N_DEV,)),
        ],
        compiler_params=pltpu.CompilerParams(collective_id=0),
    )(x, g2)


# baseline (device time: 53230 ns/iter reference)
import jax
import jax.numpy as jnp
from jax import lax
from jax.experimental import pallas as pl
from jax.experimental.pallas import tpu as pltpu

N_DEV = 4
EPS = 1e-5
BLK = 1024


def kernel(x, gamma):
    m, n_local = x.shape
    n_global = n_local * N_DEV
    rows = m // 128
    grid = m // BLK
    tb = BLK // 128
    g2 = gamma.reshape(1, n_local)

    def body_a(x_ref, inv_ref, comm_ref, send_sems, recv_sems):
        i = pl.program_id(0)
        my = lax.axis_index("i")

        @pl.when(i == 0)
        def _barrier():
            barrier = pltpu.get_barrier_semaphore()
            for k in range(1, N_DEV):
                peer = lax.rem(my + k, N_DEV)
                pl.semaphore_signal(
                    barrier, inc=1,
                    device_id=(peer,), device_id_type=pl.DeviceIdType.MESH,
                )
            pl.semaphore_wait(barrier, N_DEV - 1)

        x3 = x_ref[:, :].reshape(tb, 128, n_local)
        comm_ref[0, pl.ds(i * tb, tb), :] = jnp.sum(x3 * x3, axis=2)

        @pl.when(i == grid - 1)
        def _exchange():
            rdmas = []
            for k in range(1, N_DEV):
                peer = lax.rem(my + k, N_DEV)
                rdma = pltpu.make_async_remote_copy(
                    src_ref=comm_ref.at[0],
                    dst_ref=comm_ref.at[N_DEV - k],
                    send_sem=send_sems.at[k - 1],
                    recv_sem=recv_sems.at[N_DEV - k],
                    device_id=(peer,),
                    device_id_type=pl.DeviceIdType.MESH,
                )
                rdma.start()
                rdmas.append(rdma)
            for rdma in rdmas:
                rdma.wait()
            total = (
                comm_ref[0, :, :] + comm_ref[1, :, :]
                + comm_ref[2, :, :] + comm_ref[3, :, :]
            )
            inv_ref[:, :] = lax.rsqrt(total / n_global + EPS)

    inv = pl.pallas_call(
        body_a,
        grid=(grid,),
        out_shape=jax.ShapeDtypeStruct((rows, 128), jnp.float32),
        in_specs=[pl.BlockSpec((BLK, n_local), lambda i: (i, 0))],
        out_specs=pl.BlockSpec((rows, 128), lambda i: (0, 0)),
        scratch_shapes=[
            pltpu.VMEM((N_DEV, rows, 128), jnp.float32),
            pltpu.SemaphoreType.DMA((N_DEV - 1,)),
            pltpu.SemaphoreType.DMA((N_DEV,)),
        ],
        compiler_params=pltpu.CompilerParams(collective_id=0),
    )(x)

    def body_b(x_ref, inv_ref, g_ref, out_ref):
        x3 = x_ref[:, :].reshape(tb, 128, n_local)
        s3 = inv_ref[:, :].reshape(tb, 128, 1)
        out_ref[:, :] = (x3 * s3 * g_ref[:, :]).reshape(BLK, n_local)

    return pl.pallas_call(
        body_b,
        grid=(grid,),
        out_shape=jax.ShapeDtypeStruct((m, n_local), x.dtype),
        in_specs=[
            pl.BlockSpec((BLK, n_local), lambda i: (i, 0)),
            pl.BlockSpec((tb, 128), lambda i: (i, 0)),
            pl.BlockSpec((1, n_local), lambda i: (0, 0)),
        ],
        out_specs=pl.BlockSpec((BLK, n_local), lambda i: (i, 0)),
    )(x, inv, g2)
